# Initial kernel scaffold; baseline (speedup 1.0000x reference)
import jax
import jax.numpy as jnp
from jax.experimental import pallas as pl


def kernel(x, edge_index, edge_attr, batch, params):
    raise NotImplementedError("write your pallas kernel here")


# speedup vs baseline: 3.6978x; 3.6978x over previous
import functools

import jax
import jax.numpy as jnp
from jax import lax
from jax.experimental import pallas as pl
from jax.experimental.pallas import tpu as pltpu
from jax.experimental.pallas import tpu_sc as plsc

N = 10000
E = 320000
NODE_IN = 128
EDGE_IN = 16
H = 64
L = 3
G = 64

NC = 2
NS = 16
NW = NC * NS
EPT = E // NW
CH = 80
NCHUNK = EPT // CH
ACC_N = 10240
NROW = ACC_N // NS
RB = 128


def _aggr_body(h_hbm, e_hbm, src_hbm, dst_hbm, out_hbm,
               src_v, dst_v, zbuf, hbuf, ebuf, acc, gsem):
    c = lax.axis_index("c")
    s = lax.axis_index("s")
    wid = s * NC + c

    pltpu.sync_copy(src_hbm.at[wid], src_v)
    pltpu.sync_copy(dst_hbm.at[wid], dst_v)

    zeros16 = jnp.zeros((16,), jnp.float32)

    @pl.loop(0, RB)
    def _zero_row(r):
        for cc in range(H // 16):
            zbuf[r, pl.ds(cc * 16, 16)] = zeros16

    for k in range(NROW // RB):
        pltpu.sync_copy(zbuf, acc.at[pl.ds(s * NROW + k * RB, RB)])

    plsc.subcore_barrier()

    gbase = wid * NCHUNK

    @pl.loop(0, NCHUNK)
    def _chunk(j):
        pltpu.async_copy(h_hbm.at[src_v.at[j]], hbuf, gsem).wait()
        pltpu.sync_copy(e_hbm.at[pl.ds((gbase + j) * CH, CH)], ebuf)

        @pl.loop(0, CH)
        def _row(r):
            for cc in range(H // 16):
                sl = pl.ds(cc * 16, 16)
                ebuf[r, sl] = jnp.maximum(hbuf[r, sl] + ebuf[r, sl], 0.0)

        pltpu.sync_copy(ebuf, acc.at[dst_v.at[j]], add=True)

    plsc.subcore_barrier()

    for k in range(NROW // RB):
        row0 = s * NROW + k * RB
        pltpu.sync_copy(acc.at[pl.ds(row0, RB)], zbuf)
        pltpu.sync_copy(zbuf, out_hbm.at[c, pl.ds(row0, RB)])


def _make_aggr():
    return pl.kernel(
        _aggr_body,
        out_type=jax.ShapeDtypeStruct((NC, ACC_N, H), jnp.float32),
        mesh=plsc.VectorSubcoreMesh(core_axis_name="c", subcore_axis_name="s"),
        compiler_params=pltpu.CompilerParams(use_tc_tiling_on_sc=False),
        scratch_types=[
            pltpu.VMEM((NCHUNK, CH), jnp.int32),
            pltpu.VMEM((NCHUNK, CH), jnp.int32),
            pltpu.VMEM((RB, H), jnp.float32),
            pltpu.VMEM((CH, H), jnp.float32),
            pltpu.VMEM((CH, H), jnp.float32),
            pltpu.VMEM_SHARED((ACC_N, H), jnp.float32),
            pltpu.SemaphoreType.DMA,
        ],
    )


def _node_enc_body(x_ref, w_ref, b_ref, o_ref):
    o_ref[...] = (
        jnp.dot(x_ref[...], w_ref[...], preferred_element_type=jnp.float32)
        + b_ref[...]
    )


def _edge_enc_body(a_ref, w_ref, b_ref, o_ref):
    o_ref[...] = (
        jnp.dot(a_ref[...], w_ref[...], preferred_element_type=jnp.float32)
        + b_ref[...]
    )


def _layer_body(eps_ref, h_ref, a_ref, w1_ref, b1_ref, w2_ref,
                b2_ref, g_ref, be_ref, o_ref):
    scale = 1.0 + eps_ref[0]
    z = scale * h_ref[...] + a_ref[0, :N, :] + a_ref[1, :N, :]
    z = jnp.maximum(
        jnp.dot(z, w1_ref[...], preferred_element_type=jnp.float32)
        + b1_ref[...], 0.0)
    z = (jnp.dot(z, w2_ref[...], preferred_element_type=jnp.float32)
         + b2_ref[...])
    mean = jnp.mean(z, axis=0, keepdims=True)
    var = jnp.mean((z - mean) ** 2, axis=0, keepdims=True)
    z = (z - mean) * lax.rsqrt(var + 1e-5) * g_ref[...] + be_ref[...]
    o_ref[...] = jnp.maximum(z, 0.0)


def _head_body(h_ref, b_ref, w1_ref, b1_ref, w2_ref, b2_ref, o_ref):
    gid = lax.broadcasted_iota(jnp.int32, (G, 1), 0)
    onehot_t = (gid == b_ref[...]).astype(jnp.float32)
    sums = jnp.dot(onehot_t, h_ref[...],
                   preferred_element_type=jnp.float32)
    counts = jnp.sum(onehot_t, axis=1, keepdims=True)
    g = sums / jnp.maximum(counts, 1.0)
    gh = jnp.maximum(
        jnp.dot(g, w1_ref[...], preferred_element_type=jnp.float32)
        + b1_ref[...], 0.0)
    o_ref[...] = (
        jnp.dot(gh, w2_ref[...], preferred_element_type=jnp.float32)
        + b2_ref[...])


def _tc_call(body, out_shape, num_inputs, smem_first=False):
    specs = []
    if smem_first:
        specs.append(pl.BlockSpec(memory_space=pltpu.SMEM))
        num_inputs -= 1
    specs.extend([pl.BlockSpec(memory_space=pltpu.VMEM)] * num_inputs)
    return pl.pallas_call(
        body,
        in_specs=specs,
        out_specs=pl.BlockSpec(memory_space=pltpu.VMEM),
        out_shape=out_shape,
    )


def kernel(x, edge_index, edge_attr, batch, params):
    src = edge_index[0].reshape(NW, NCHUNK, CH)
    dst = edge_index[1].reshape(NW, NCHUNK, CH)

    h = _tc_call(_node_enc_body, jax.ShapeDtypeStruct((N, H), jnp.float32), 3)(
        x, params["node_enc"]["W"], params["node_enc"]["b"].reshape(1, H))

    BE = 16000
    e = pl.pallas_call(
        _edge_enc_body,
        grid=(E // BE,),
        in_specs=[
            pl.BlockSpec((BE, EDGE_IN), lambda i: (i, 0)),
            pl.BlockSpec((EDGE_IN, H), lambda i: (0, 0)),
            pl.BlockSpec((1, H), lambda i: (0, 0)),
        ],
        out_specs=pl.BlockSpec((BE, H), lambda i: (i, 0)),
        out_shape=jax.ShapeDtypeStruct((E, H), jnp.float32),
    )(edge_attr, params["edge_enc"]["W"], params["edge_enc"]["b"].reshape(1, H))

    aggr_fn = _make_aggr()
    layer_fn = _tc_call(
        _layer_body, jax.ShapeDtypeStruct((N, H), jnp.float32), 9,
        smem_first=True)

    for lp in params["layers"]:
        aggr = aggr_fn(h, e, src, dst)
        h = layer_fn(
            lp["eps"].reshape(1),
            h, aggr,
            lp["mlp1"]["W"], lp["mlp1"]["b"].reshape(1, H),
            lp["mlp2"]["W"], lp["mlp2"]["b"].reshape(1, H),
            lp["bn_gamma"].reshape(1, H), lp["bn_beta"].reshape(1, H))

    logits2d = _tc_call(
        _head_body, jax.ShapeDtypeStruct((G, 1), jnp.float32), 6)(
            h, batch.reshape(1, N),
            params["head1"]["W"], params["head1"]["b"].reshape(1, H),
            params["head2"]["W"], params["head2"]["b"].reshape(1, 1))
    return logits2d.reshape(-1)


# speedup vs baseline: 6.4219x; 1.7367x over previous
import functools

import jax
import jax.numpy as jnp
from jax import lax
from jax.experimental import pallas as pl
from jax.experimental.pallas import tpu as pltpu
from jax.experimental.pallas import tpu_sc as plsc

N = 10000
E = 320000
NODE_IN = 128
EDGE_IN = 16
H = 64
L = 3
G = 64

NC = 2
NS = 16
NW = NC * NS
EPT = E // NW
CH = 100
NCHUNK = EPT // CH
ACC_N = 10240
NROW = ACC_N // NS
RB = 128


def _aggr_body(h_hbm, e_hbm, src_hbm, dst_hbm, out_hbm,
               src_v, dst_v, zbuf, hbuf, ebuf, acc,
               gsem0, gsem1, esem0, esem1, ssem0, ssem1):
    c = lax.axis_index("c")
    s = lax.axis_index("s")
    wid = s * NC + c
    gsem = (gsem0, gsem1)
    esem = (esem0, esem1)
    ssem = (ssem0, ssem1)

    pltpu.sync_copy(src_hbm.at[wid], src_v)
    pltpu.sync_copy(dst_hbm.at[wid], dst_v)

    zeros16 = jnp.zeros((16,), jnp.float32)

    @pl.loop(0, RB)
    def _zero_row(r):
        for cc in range(H // 16):
            zbuf[r, pl.ds(cc * 16, 16)] = zeros16

    for k in range(NROW // RB):
        pltpu.sync_copy(zbuf, acc.at[pl.ds(s * NROW + k * RB, RB)])

    plsc.subcore_barrier()

    gbase = wid * NCHUNK

    def issue(j, b):
        pltpu.async_copy(h_hbm.at[src_v.at[j]], hbuf.at[b], gsem[b])
        pltpu.async_copy(e_hbm.at[pl.ds((gbase + j) * CH, CH)],
                         ebuf.at[b], esem[b])

    def wait_in(j, b):
        pltpu.make_async_copy(h_hbm.at[src_v.at[j]], hbuf.at[b],
                              gsem[b]).wait()
        pltpu.make_async_copy(e_hbm.at[pl.ds((gbase + j) * CH, CH)],
                              ebuf.at[b], esem[b]).wait()

    def compute(b):
        @pl.loop(0, CH)
        def _row(r):
            for cc in range(H // 16):
                sl = pl.ds(cc * 16, 16)
                ebuf[b, r, sl] = jnp.maximum(
                    hbuf[b, r, sl] + ebuf[b, r, sl], 0.0)

    def scatter(j, b):
        return pltpu.async_copy(ebuf.at[b], acc.at[dst_v.at[j]], ssem[b],
                                add=True)

    issue(0, 0)
    issue(1, 1)

    @pl.loop(0, NCHUNK // 2 - 1)
    def _pair(i):
        j0 = 2 * i
        for b in range(2):
            j = j0 + b
            wait_in(j, b)
            compute(b)
            scatter(j, b).wait()
            issue(j + 2, b)

    for b in range(2):
        j = NCHUNK - 2 + b
        wait_in(j, b)
        compute(b)
        scatter(j, b).wait()

    plsc.subcore_barrier()

    for k in range(NROW // RB):
        row0 = s * NROW + k * RB
        pltpu.sync_copy(acc.at[pl.ds(row0, RB)], zbuf)
        pltpu.sync_copy(zbuf, out_hbm.at[c, pl.ds(row0, RB)])


def _make_aggr():
    return pl.kernel(
        _aggr_body,
        out_type=jax.ShapeDtypeStruct((NC, ACC_N, H), jnp.float32),
        mesh=plsc.VectorSubcoreMesh(core_axis_name="c", subcore_axis_name="s"),
        compiler_params=pltpu.CompilerParams(use_tc_tiling_on_sc=False),
        scratch_types=[
            pltpu.VMEM((NCHUNK, CH), jnp.int32),
            pltpu.VMEM((NCHUNK, CH), jnp.int32),
            pltpu.VMEM((RB, H), jnp.float32),
            pltpu.VMEM((2, CH, H), jnp.float32),
            pltpu.VMEM((2, CH, H), jnp.float32),
            pltpu.VMEM_SHARED((ACC_N, H), jnp.float32),
            pltpu.SemaphoreType.DMA,
            pltpu.SemaphoreType.DMA,
            pltpu.SemaphoreType.DMA,
            pltpu.SemaphoreType.DMA,
            pltpu.SemaphoreType.DMA,
            pltpu.SemaphoreType.DMA,
        ],
    )


def _node_enc_body(x_ref, w_ref, b_ref, o_ref):
    o_ref[...] = (
        jnp.dot(x_ref[...], w_ref[...], preferred_element_type=jnp.float32)
        + b_ref[...]
    )


def _edge_enc_body(a_ref, w_ref, b_ref, o_ref):
    o_ref[...] = (
        jnp.dot(a_ref[...], w_ref[...], preferred_element_type=jnp.float32)
        + b_ref[...]
    )


def _layer_body(eps_ref, h_ref, a_ref, w1_ref, b1_ref, w2_ref,
                b2_ref, g_ref, be_ref, o_ref):
    scale = 1.0 + eps_ref[0]
    z = scale * h_ref[...] + a_ref[0, :N, :] + a_ref[1, :N, :]
    z = jnp.maximum(
        jnp.dot(z, w1_ref[...], preferred_element_type=jnp.float32)
        + b1_ref[...], 0.0)
    z = (jnp.dot(z, w2_ref[...], preferred_element_type=jnp.float32)
         + b2_ref[...])
    mean = jnp.mean(z, axis=0, keepdims=True)
    var = jnp.mean((z - mean) ** 2, axis=0, keepdims=True)
    z = (z - mean) * lax.rsqrt(var + 1e-5) * g_ref[...] + be_ref[...]
    o_ref[...] = jnp.maximum(z, 0.0)


def _head_body(h_ref, b_ref, w1_ref, b1_ref, w2_ref, b2_ref, o_ref):
    gid = lax.broadcasted_iota(jnp.int32, (G, 1), 0)
    onehot_t = (gid == b_ref[...]).astype(jnp.float32)
    sums = jnp.dot(onehot_t, h_ref[...],
                   preferred_element_type=jnp.float32)
    counts = jnp.sum(onehot_t, axis=1, keepdims=True)
    g = sums / jnp.maximum(counts, 1.0)
    gh = jnp.maximum(
        jnp.dot(g, w1_ref[...], preferred_element_type=jnp.float32)
        + b1_ref[...], 0.0)
    o_ref[...] = (
        jnp.dot(gh, w2_ref[...], preferred_element_type=jnp.float32)
        + b2_ref[...])


def _tc_call(body, out_shape, num_inputs, smem_first=False):
    specs = []
    if smem_first:
        specs.append(pl.BlockSpec(memory_space=pltpu.SMEM))
        num_inputs -= 1
    specs.extend([pl.BlockSpec(memory_space=pltpu.VMEM)] * num_inputs)
    return pl.pallas_call(
        body,
        in_specs=specs,
        out_specs=pl.BlockSpec(memory_space=pltpu.VMEM),
        out_shape=out_shape,
    )


def kernel(x, edge_index, edge_attr, batch, params):
    src = edge_index[0].reshape(NW, NCHUNK, CH)
    dst = edge_index[1].reshape(NW, NCHUNK, CH)

    h = _tc_call(_node_enc_body, jax.ShapeDtypeStruct((N, H), jnp.float32), 3)(
        x, params["node_enc"]["W"], params["node_enc"]["b"].reshape(1, H))

    BE = 16000
    e = pl.pallas_call(
        _edge_enc_body,
        grid=(E // BE,),
        in_specs=[
            pl.BlockSpec((BE, EDGE_IN), lambda i: (i, 0)),
            pl.BlockSpec((EDGE_IN, H), lambda i: (0, 0)),
            pl.BlockSpec((1, H), lambda i: (0, 0)),
        ],
        out_specs=pl.BlockSpec((BE, H), lambda i: (i, 0)),
        out_shape=jax.ShapeDtypeStruct((E, H), jnp.float32),
    )(edge_attr, params["edge_enc"]["W"], params["edge_enc"]["b"].reshape(1, H))

    aggr_fn = _make_aggr()
    layer_fn = _tc_call(
        _layer_body, jax.ShapeDtypeStruct((N, H), jnp.float32), 9,
        smem_first=True)

    for lp in params["layers"]:
        aggr = aggr_fn(h, e, src, dst)
        h = layer_fn(
            lp["eps"].reshape(1),
            h, aggr,
            lp["mlp1"]["W"], lp["mlp1"]["b"].reshape(1, H),
            lp["mlp2"]["W"], lp["mlp2"]["b"].reshape(1, H),
            lp["bn_gamma"].reshape(1, H), lp["bn_beta"].reshape(1, H))

    logits2d = _tc_call(
        _head_body, jax.ShapeDtypeStruct((G, 1), jnp.float32), 6)(
            h, batch.reshape(1, N),
            params["head1"]["W"], params["head1"]["b"].reshape(1, H),
            params["head2"]["W"], params["head2"]["b"].reshape(1, 1))
    return logits2d.reshape(-1)
